# Initial kernel scaffold; baseline (speedup 1.0000x reference)
import jax
import jax.numpy as jnp
from jax.experimental import pallas as pl


def kernel(node_x, in_degree, out_degree, edge_index, edge_attr, batch, node_enc, in_deg_enc, out_deg_enc, ln_g, ln_b, wl_W, wl_b, mlp_W, mlp_b, r0_W, r0_b, r1_W, r1_b, r2_W, r2_b, r3_W, r3_b):
    raise NotImplementedError("write your pallas kernel here")


# speedup vs baseline: 1.8686x; 1.8686x over previous
import jax
import jax.numpy as jnp
from jax.experimental import pallas as pl
from jax.experimental.pallas import tpu as pltpu

N = 10000
E = 320000
D = 128
L = 6
G = 64


def _layer_norm(x, g, b, eps=1e-5):
    mu = jnp.mean(x, axis=-1, keepdims=True)
    var = jnp.mean((x - mu) ** 2, axis=-1, keepdims=True)
    return (x - mu) / jnp.sqrt(var + eps) * g + b


def _gelu(x):
    return 0.5 * x * (1.0 + jax.lax.erf(x * 0.7071067811865476))


def _readout_kernel(h_ref, w0, b0, w1, b1, w2, b2, w3, b3, out_ref):
    h = h_ref[...]
    h = _gelu(h @ w0[...] + b0[...][None, :])
    h = _gelu(h @ w1[...] + b1[...][None, :])
    h = _gelu(h @ w2[...] + b2[...][None, :])
    out_ref[...] = h @ w3[...] + b3[...][None, :]


def kernel(node_x, in_degree, out_degree, edge_index, edge_attr, batch,
           node_enc, in_deg_enc, out_deg_enc, ln_g, ln_b, wl_W, wl_b,
           mlp_W, mlp_b, r0_W, r0_b, r1_W, r1_b, r2_W, r2_b, r3_W, r3_b):
    x = jnp.take(node_enc, node_x, axis=0).sum(axis=-2) \
        + jnp.take(in_deg_enc, in_degree, axis=0) \
        + jnp.take(out_deg_enc, out_degree, axis=0)
    src = edge_index[0]
    dst = edge_index[1]
    ea = edge_attr[:, None]
    outs = []
    for l in range(L):
        y = _layer_norm(x, ln_g[l], ln_b[l])
        edge_emb = ea * wl_W[l, 0][None, :] + wl_b[l][None, :]
        msg = jax.nn.relu(jnp.take(y, src, axis=0) + edge_emb) + 1e-7
        ex = jnp.exp(msg)
        den = jax.ops.segment_sum(ex, dst, num_segments=N)
        num = jax.ops.segment_sum(msg * ex, dst, num_segments=N)
        aggr = num / (den + 1e-16)
        out = (aggr + y) @ mlp_W[l] + mlp_b[l]
        x = jax.nn.relu(out)
        outs.append(jax.ops.segment_sum(x, batch, num_segments=G))
    h = jnp.concatenate(outs, axis=1)

    out = pl.pallas_call(
        _readout_kernel,
        out_shape=jax.ShapeDtypeStruct((G, 1), jnp.float32),
    )(h, r0_W, r0_b, r1_W, r1_b, r2_W, r2_b, r3_W, r3_b)
    return out


# speedup vs baseline: 2.2783x; 1.2193x over previous
import functools

import jax
import jax.numpy as jnp
from jax import lax
from jax.experimental import pallas as pl
from jax.experimental.pallas import tpu as pltpu
from jax.experimental.pallas import tpu_sc as plsc

N = 10000
E = 320000
D = 128
L = 6
G = 64

NS = 16
NC = 2
CHUNK = 128
CHUNKS = 158
EP = CHUNKS * CHUNK * NS
ROWS_PER_TILE = 624


def _layer_norm(x, g, b, eps=1e-5):
    mu = jnp.mean(x, axis=-1, keepdims=True)
    var = jnp.mean((x - mu) ** 2, axis=-1, keepdims=True)
    return (x - mu) / jnp.sqrt(var + eps) * g + b


def _gelu(x):
    return 0.5 * x * (1.0 + jax.lax.erf(x * 0.7071067811865476))


def _readout_kernel(h_ref, w0, b0, w1, b1, w2, b2, w3, b3, out_ref):
    h = h_ref[...]
    h = _gelu(h @ w0[...] + b0[...][None, :])
    h = _gelu(h @ w1[...] + b1[...][None, :])
    h = _gelu(h @ w2[...] + b2[...][None, :])
    out_ref[...] = h @ w3[...] + b3[...][None, :]


def _edge_body(y2, srcp, dstp, eap, wb, acc_out,
               shared_acc, idxb, dstb, eab, rows, outb, wbv, sem):
    c = lax.axis_index("c")
    s = lax.axis_index("s")
    pltpu.sync_copy(wb.at[c], wbv)

    z16 = jnp.zeros((16,), jnp.float32)

    def zloop(t, carry):
        outb[t, pl.ds(0, 16)] = z16
        outb[t, pl.ds(16, 16)] = z16
        outb[t, pl.ds(32, 16)] = z16
        outb[t, pl.ds(48, 16)] = z16
        outb[t, pl.ds(64, 16)] = z16
        outb[t, pl.ds(80, 16)] = z16
        outb[t, pl.ds(96, 16)] = z16
        outb[t, pl.ds(112, 16)] = z16
        return carry

    lax.fori_loop(0, 128, zloop, 0)
    zb = s * ROWS_PER_TILE
    for k in range(4):
        pltpu.sync_copy(outb.at[pl.ds(0, 128)],
                        shared_acc.at[pl.ds(zb + k * 128, 128)])
    pltpu.sync_copy(outb.at[pl.ds(0, ROWS_PER_TILE - 512)],
                    shared_acc.at[pl.ds(zb + 512, ROWS_PER_TILE - 512)])

    @pl.when(s == NS - 1)
    def _zero_tail():
        pltpu.sync_copy(outb.at[pl.ds(0, 32)],
                        shared_acc.at[pl.ds(NS * ROWS_PER_TILE, 32)])

    plsc.subcore_barrier()

    cN = c * N

    def chunk_body(i, carry):
        base = (s * CHUNKS + i) * CHUNK
        pltpu.sync_copy(srcp.at[pl.ds(base, CHUNK)], idxb)
        pltpu.sync_copy(dstp.at[pl.ds(base, CHUNK)], dstb)
        pltpu.sync_copy(eap.at[pl.ds(base, CHUNK)], eab.at[pl.ds(0, CHUNK)])
        for j in range(8):
            idxb[pl.ds(j * 16, 16)] = idxb[pl.ds(j * 16, 16)] + cN
        pltpu.async_copy(y2.at[idxb], rows, sem).wait()

        def edge_body(e, ecarry):
            ea_s = eab[pl.ds(e, 16)][0]
            for v in range(4):
                yv = rows[e, pl.ds(v * 16, 16)]
                wv = wbv[0, pl.ds(v * 16, 16)]
                bv = wbv[1, pl.ds(v * 16, 16)]
                m = jnp.maximum(yv + ea_s * wv + bv, 0.0) + 1e-7
                ex = jnp.exp(m)
                outb[e, pl.ds(v * 16, 16)] = m * ex
                outb[e, pl.ds(64 + v * 16, 16)] = ex
            return ecarry

        lax.fori_loop(0, CHUNK, edge_body, 0)
        pltpu.sync_copy(outb, shared_acc.at[dstb], add=True)
        return carry

    lax.fori_loop(0, CHUNKS, chunk_body, 0)
    plsc.subcore_barrier()
    rb = s * ROWS_PER_TILE
    pltpu.sync_copy(shared_acc.at[pl.ds(rb, ROWS_PER_TILE)],
                    acc_out.at[c, pl.ds(rb, ROWS_PER_TILE)])

    @pl.when(s == NS - 1)
    def _copy_tail():
        pltpu.sync_copy(shared_acc.at[pl.ds(NS * ROWS_PER_TILE, 16)],
                        acc_out.at[c, pl.ds(NS * ROWS_PER_TILE, 16)])


_edge_pass = functools.partial(
    pl.kernel,
    out_type=jax.ShapeDtypeStruct((NC, N, 128), jnp.float32),
    mesh=plsc.VectorSubcoreMesh(core_axis_name="c", subcore_axis_name="s"),
    compiler_params=pltpu.CompilerParams(use_tc_tiling_on_sc=False),
    scratch_types=[
        pltpu.VMEM_SHARED((N + 16, 128), jnp.float32),
        pltpu.VMEM((CHUNK,), jnp.int32),
        pltpu.VMEM((CHUNK,), jnp.int32),
        pltpu.VMEM((CHUNK + 16,), jnp.float32),
        pltpu.VMEM((CHUNK, 64), jnp.float32),
        pltpu.VMEM((CHUNK, 128), jnp.float32),
        pltpu.VMEM((2, 64), jnp.float32),
        pltpu.SemaphoreType.DMA,
    ],
)(_edge_body)


def kernel(node_x, in_degree, out_degree, edge_index, edge_attr, batch,
           node_enc, in_deg_enc, out_deg_enc, ln_g, ln_b, wl_W, wl_b,
           mlp_W, mlp_b, r0_W, r0_b, r1_W, r1_b, r2_W, r2_b, r3_W, r3_b):
    x = jnp.take(node_enc, node_x, axis=0).sum(axis=-2) \
        + jnp.take(in_deg_enc, in_degree, axis=0) \
        + jnp.take(out_deg_enc, out_degree, axis=0)

    pad = EP - E
    srcp = jnp.concatenate(
        [edge_index[0].astype(jnp.int32), jnp.zeros((pad,), jnp.int32)])
    dstp = jnp.concatenate(
        [edge_index[1].astype(jnp.int32), jnp.full((pad,), N, jnp.int32)])
    eap = jnp.concatenate([edge_attr, jnp.zeros((pad,), jnp.float32)])

    outs = []
    for l in range(L):
        y = _layer_norm(x, ln_g[l], ln_b[l])
        y2 = jnp.concatenate([y[:, :64], y[:, 64:]], axis=0)
        wb = jnp.stack([
            jnp.stack([wl_W[l, 0, :64], wl_b[l, :64]]),
            jnp.stack([wl_W[l, 0, 64:], wl_b[l, 64:]]),
        ])
        acc = _edge_pass(y2, srcp, dstp, eap, wb)
        num = jnp.concatenate([acc[0, :, :64], acc[1, :, :64]], axis=1)
        den = jnp.concatenate([acc[0, :, 64:], acc[1, :, 64:]], axis=1)
        aggr = num / (den + 1e-16)
        out = (aggr + y) @ mlp_W[l] + mlp_b[l]
        x = jax.nn.relu(out)
        outs.append(jax.ops.segment_sum(x, batch, num_segments=G))
    h = jnp.concatenate(outs, axis=1)

    out = pl.pallas_call(
        _readout_kernel,
        out_shape=jax.ShapeDtypeStruct((G, 1), jnp.float32),
    )(h, r0_W, r0_b, r1_W, r1_b, r2_W, r2_b, r3_W, r3_b)
    return out


# speedup vs baseline: 3.1430x; 1.3795x over previous
import functools

import jax
import jax.numpy as jnp
from jax import lax
from jax.experimental import pallas as pl
from jax.experimental.pallas import tpu as pltpu
from jax.experimental.pallas import tpu_sc as plsc

N = 10000
E = 320000
D = 128
L = 6
G = 64

NS = 16
NC = 2
CHUNK = 128
CHUNKS = 160
EP = CHUNKS * CHUNK * NS
ROWS_PER_TILE = 624


def _layer_norm(x, g, b, eps=1e-5):
    mu = jnp.mean(x, axis=-1, keepdims=True)
    var = jnp.mean((x - mu) ** 2, axis=-1, keepdims=True)
    return (x - mu) / jnp.sqrt(var + eps) * g + b


def _gelu(x):
    return 0.5 * x * (1.0 + jax.lax.erf(x * 0.7071067811865476))


def _readout_kernel(h_ref, w0, b0, w1, b1, w2, b2, w3, b3, out_ref):
    h = h_ref[...]
    h = _gelu(h @ w0[...] + b0[...][None, :])
    h = _gelu(h @ w1[...] + b1[...][None, :])
    h = _gelu(h @ w2[...] + b2[...][None, :])
    out_ref[...] = h @ w3[...] + b3[...][None, :]


def _edge_body(y2, srcp, dstp, eap, wb, acc_out,
               shared_acc, idxb, dstb, eab, rows, outb, wbv,
               semg, sems, seml):
    c = lax.axis_index("c")
    s = lax.axis_index("s")
    pltpu.sync_copy(wb.at[c], wbv)
    cN = c * N

    def lin_issue(j, slot):
        base = (s * CHUNKS + j) * CHUNK
        pltpu.async_copy(srcp.at[pl.ds(base, CHUNK)], idxb.at[slot],
                         seml.at[slot])
        pltpu.async_copy(dstp.at[pl.ds(base, CHUNK)], dstb.at[slot],
                         seml.at[slot])
        pltpu.async_copy(eap.at[pl.ds(base, CHUNK)],
                         eab.at[slot, pl.ds(0, CHUNK)], seml.at[slot])

    def lin_wait_and_gather(j, slot):
        base = (s * CHUNKS + j) * CHUNK
        pltpu.make_async_copy(srcp.at[pl.ds(base, CHUNK)], idxb.at[slot],
                              seml.at[slot]).wait()
        pltpu.make_async_copy(dstp.at[pl.ds(base, CHUNK)], dstb.at[slot],
                              seml.at[slot]).wait()
        pltpu.make_async_copy(eap.at[pl.ds(base, CHUNK)],
                              eab.at[slot, pl.ds(0, CHUNK)],
                              seml.at[slot]).wait()
        for t in range(8):
            idxb[slot, pl.ds(t * 16, 16)] = \
                idxb[slot, pl.ds(t * 16, 16)] + cN
        pltpu.async_copy(y2.at[idxb.at[slot]], rows.at[slot % 2],
                         semg.at[slot])

    for k in range(2):
        lin_issue(k, k)

    z16 = jnp.zeros((16,), jnp.float32)

    def zloop(t, carry):
        for v in range(8):
            outb[0, t, pl.ds(v * 16, 16)] = z16
        return carry

    lax.fori_loop(0, 128, zloop, 0)
    zb = s * ROWS_PER_TILE
    for k in range(4):
        pltpu.sync_copy(outb.at[0, pl.ds(0, 128)],
                        shared_acc.at[pl.ds(zb + k * 128, 128)])
    pltpu.sync_copy(outb.at[0, pl.ds(0, ROWS_PER_TILE - 512)],
                    shared_acc.at[pl.ds(zb + 512, ROWS_PER_TILE - 512)])

    @pl.when(s == NS - 1)
    def _zero_tail():
        pltpu.sync_copy(outb.at[0, pl.ds(0, 32)],
                        shared_acc.at[pl.ds(NS * ROWS_PER_TILE, 32)])

    for k in range(2):
        lin_wait_and_gather(k, k)

    plsc.subcore_barrier()

    def scatter_wait(slot):
        pltpu.make_async_copy(outb.at[slot % 2],
                              shared_acc.at[dstb.at[slot]],
                              sems.at[slot]).wait()

    w0 = wbv[0, pl.ds(0, 16)]
    w1 = wbv[0, pl.ds(16, 16)]
    w2 = wbv[0, pl.ds(32, 16)]
    w3 = wbv[0, pl.ds(48, 16)]
    b0 = wbv[1, pl.ds(0, 16)]
    b1 = wbv[1, pl.ds(16, 16)]
    b2 = wbv[1, pl.ds(32, 16)]
    b3 = wbv[1, pl.ds(48, 16)]

    def group_body(g, carry):
        for k in range(4):
            i = 4 * g + k
            slot = k
            k4 = k % 2
            if k >= 2:
                scatter_wait(k - 2)
            else:
                @pl.when(g > 0)
                def _w():
                    scatter_wait(k + 2)
            pltpu.make_async_copy(y2.at[idxb.at[slot]], rows.at[k4],
                                  semg.at[slot]).wait()
            kp = (k + 2) % 4
            if k < 2:
                lin_issue(i + 2, kp)
            else:
                @pl.when(g < (CHUNKS // 4) - 1)
                def _p():
                    lin_issue(i + 2, kp)

            def edge_body(e, ecarry):
                cw0, cw1, cw2, cw3, cb0, cb1, cb2, cb3 = ecarry
                ea_s = eab[slot, pl.ds(e, 16)][0]
                yv = rows[k4, e, pl.ds(0, 16)]
                m = jnp.maximum(yv + ea_s * cw0 + cb0, 0.0) + 1e-7
                ex = jnp.exp(m)
                outb[k4, e, pl.ds(0, 16)] = m * ex
                outb[k4, e, pl.ds(64, 16)] = ex
                yv = rows[k4, e, pl.ds(16, 16)]
                m = jnp.maximum(yv + ea_s * cw1 + cb1, 0.0) + 1e-7
                ex = jnp.exp(m)
                outb[k4, e, pl.ds(16, 16)] = m * ex
                outb[k4, e, pl.ds(80, 16)] = ex
                yv = rows[k4, e, pl.ds(32, 16)]
                m = jnp.maximum(yv + ea_s * cw2 + cb2, 0.0) + 1e-7
                ex = jnp.exp(m)
                outb[k4, e, pl.ds(32, 16)] = m * ex
                outb[k4, e, pl.ds(96, 16)] = ex
                yv = rows[k4, e, pl.ds(48, 16)]
                m = jnp.maximum(yv + ea_s * cw3 + cb3, 0.0) + 1e-7
                ex = jnp.exp(m)
                outb[k4, e, pl.ds(48, 16)] = m * ex
                outb[k4, e, pl.ds(112, 16)] = ex
                return ecarry

            lax.fori_loop(0, CHUNK, edge_body,
                          (w0, w1, w2, w3, b0, b1, b2, b3))
            pltpu.async_copy(outb.at[k4], shared_acc.at[dstb.at[slot]],
                             sems.at[slot], add=True)
            if k < 2:
                lin_wait_and_gather(i + 2, kp)
            else:
                @pl.when(g < (CHUNKS // 4) - 1)
                def _q():
                    lin_wait_and_gather(i + 2, kp)
        return carry

    lax.fori_loop(0, CHUNKS // 4, group_body, 0)
    for k in range(2, 4):
        scatter_wait(k)
    plsc.subcore_barrier()
    rb = s * ROWS_PER_TILE
    pltpu.sync_copy(shared_acc.at[pl.ds(rb, ROWS_PER_TILE)],
                    acc_out.at[c, pl.ds(rb, ROWS_PER_TILE)])

    @pl.when(s == NS - 1)
    def _copy_tail():
        pltpu.sync_copy(shared_acc.at[pl.ds(NS * ROWS_PER_TILE, 16)],
                        acc_out.at[c, pl.ds(NS * ROWS_PER_TILE, 16)])


_edge_pass = functools.partial(
    pl.kernel,
    out_type=jax.ShapeDtypeStruct((NC, N, 128), jnp.float32),
    mesh=plsc.VectorSubcoreMesh(core_axis_name="c", subcore_axis_name="s"),
    compiler_params=pltpu.CompilerParams(use_tc_tiling_on_sc=False),
    scratch_types=[
        pltpu.VMEM_SHARED((N + 16, 128), jnp.float32),
        pltpu.VMEM((4, CHUNK), jnp.int32),
        pltpu.VMEM((4, CHUNK), jnp.int32),
        pltpu.VMEM((4, CHUNK + 16), jnp.float32),
        pltpu.VMEM((2, CHUNK, 64), jnp.float32),
        pltpu.VMEM((2, CHUNK, 128), jnp.float32),
        pltpu.VMEM((2, 64), jnp.float32),
        pltpu.SemaphoreType.DMA((4,)),
        pltpu.SemaphoreType.DMA((4,)),
        pltpu.SemaphoreType.DMA((4,)),
    ],
)(_edge_body)


def kernel(node_x, in_degree, out_degree, edge_index, edge_attr, batch,
           node_enc, in_deg_enc, out_deg_enc, ln_g, ln_b, wl_W, wl_b,
           mlp_W, mlp_b, r0_W, r0_b, r1_W, r1_b, r2_W, r2_b, r3_W, r3_b):
    x = jnp.take(node_enc, node_x, axis=0).sum(axis=-2) \
        + jnp.take(in_deg_enc, in_degree, axis=0) \
        + jnp.take(out_deg_enc, out_degree, axis=0)

    pad = EP - E
    srcp = jnp.concatenate(
        [edge_index[0].astype(jnp.int32), jnp.zeros((pad,), jnp.int32)])
    dstp = jnp.concatenate(
        [edge_index[1].astype(jnp.int32), jnp.full((pad,), N, jnp.int32)])
    eap = jnp.concatenate([edge_attr, jnp.zeros((pad,), jnp.float32)])

    outs = []
    for l in range(L):
        y = _layer_norm(x, ln_g[l], ln_b[l])
        y2 = jnp.concatenate([y[:, :64], y[:, 64:]], axis=0)
        wb = jnp.stack([
            jnp.stack([wl_W[l, 0, :64], wl_b[l, :64]]),
            jnp.stack([wl_W[l, 0, 64:], wl_b[l, 64:]]),
        ])
        acc = _edge_pass(y2, srcp, dstp, eap, wb)
        num = jnp.concatenate([acc[0, :, :64], acc[1, :, :64]], axis=1)
        den = jnp.concatenate([acc[0, :, 64:], acc[1, :, 64:]], axis=1)
        aggr = num / (den + 1e-16)
        out = (aggr + y) @ mlp_W[l] + mlp_b[l]
        x = jax.nn.relu(out)
        outs.append(jax.ops.segment_sum(x, batch, num_segments=G))
    h = jnp.concatenate(outs, axis=1)

    out = pl.pallas_call(
        _readout_kernel,
        out_shape=jax.ShapeDtypeStruct((G, 1), jnp.float32),
    )(h, r0_W, r0_b, r1_W, r1_b, r2_W, r2_b, r3_W, r3_b)
    return out


# speedup vs baseline: 3.6502x; 1.1614x over previous
import functools

import jax
import jax.numpy as jnp
from jax import lax
from jax.experimental import pallas as pl
from jax.experimental.pallas import tpu as pltpu
from jax.experimental.pallas import tpu_sc as plsc

N = 10000
E = 320000
D = 128
L = 6
G = 64

NS = 16
NC = 2
CHUNK = 128
CHUNKS = 160
EP = CHUNKS * CHUNK * NS
ROWS_PER_TILE = 624


def _layer_norm(x, g, b, eps=1e-5):
    mu = jnp.mean(x, axis=-1, keepdims=True)
    var = jnp.mean((x - mu) ** 2, axis=-1, keepdims=True)
    return (x - mu) / jnp.sqrt(var + eps) * g + b


def _gelu(x):
    return 0.5 * x * (1.0 + jax.lax.erf(x * 0.7071067811865476))


def _readout_kernel(h_ref, w0, b0, w1, b1, w2, b2, w3, b3, out_ref):
    h = h_ref[...]
    h = _gelu(h @ w0[...] + b0[...][None, :])
    h = _gelu(h @ w1[...] + b1[...][None, :])
    h = _gelu(h @ w2[...] + b2[...][None, :])
    out_ref[...] = h @ w3[...] + b3[...][None, :]


def _edge_body(y2, srcp, dstp, eap, wb, acc_out,
               shared_acc, idxb, dstb, eab, rows, outb, wbv,
               semg, sems, seml):
    c = lax.axis_index("c")
    s = lax.axis_index("s")
    pltpu.sync_copy(wb.at[c], wbv)
    cN = c * N

    def lin_issue(j, slot):
        base = (s * CHUNKS + j) * CHUNK
        pltpu.async_copy(srcp.at[pl.ds(base, CHUNK)], idxb.at[slot],
                         seml.at[slot])
        pltpu.async_copy(dstp.at[pl.ds(base, CHUNK)], dstb.at[slot],
                         seml.at[slot])
        pltpu.async_copy(eap.at[pl.ds(base, CHUNK)],
                         eab.at[slot, pl.ds(0, CHUNK)], seml.at[slot])

    def lin_wait_and_gather(j, slot):
        base = (s * CHUNKS + j) * CHUNK
        pltpu.make_async_copy(srcp.at[pl.ds(base, CHUNK)], idxb.at[slot],
                              seml.at[slot]).wait()
        pltpu.make_async_copy(dstp.at[pl.ds(base, CHUNK)], dstb.at[slot],
                              seml.at[slot]).wait()
        pltpu.make_async_copy(eap.at[pl.ds(base, CHUNK)],
                              eab.at[slot, pl.ds(0, CHUNK)],
                              seml.at[slot]).wait()
        for t in range(8):
            idxb[slot, pl.ds(t * 16, 16)] = \
                idxb[slot, pl.ds(t * 16, 16)] + cN
        pltpu.async_copy(y2.at[idxb.at[slot]], rows.at[slot % 2],
                         semg.at[slot])

    for k in range(2):
        lin_issue(k, k)

    z16 = jnp.zeros((16,), jnp.float32)

    def zloop(t, carry):
        for v in range(8):
            outb[0, t, pl.ds(v * 16, 16)] = z16
        return carry

    lax.fori_loop(0, 128, zloop, 0)
    zb = s * ROWS_PER_TILE
    for k in range(4):
        pltpu.sync_copy(outb.at[0, pl.ds(0, 128)],
                        shared_acc.at[pl.ds(zb + k * 128, 128)])
    pltpu.sync_copy(outb.at[0, pl.ds(0, ROWS_PER_TILE - 512)],
                    shared_acc.at[pl.ds(zb + 512, ROWS_PER_TILE - 512)])

    @pl.when(s == NS - 1)
    def _zero_tail():
        pltpu.sync_copy(outb.at[0, pl.ds(0, 32)],
                        shared_acc.at[pl.ds(NS * ROWS_PER_TILE, 32)])

    for k in range(2):
        lin_wait_and_gather(k, k)

    plsc.subcore_barrier()

    def scatter_wait(slot):
        pltpu.make_async_copy(outb.at[slot % 2],
                              shared_acc.at[dstb.at[slot]],
                              sems.at[slot]).wait()

    w0 = wbv[pl.ds(0, 16)]
    w1 = wbv[pl.ds(16, 16)]
    w2 = wbv[pl.ds(32, 16)]
    w3 = wbv[pl.ds(48, 16)]

    def group_body(g, carry):
        for k in range(4):
            i = 4 * g + k
            slot = k
            k4 = k % 2
            if k >= 2:
                scatter_wait(k - 2)
            else:
                @pl.when(g > 0)
                def _w():
                    scatter_wait(k + 2)
            pltpu.make_async_copy(y2.at[idxb.at[slot]], rows.at[k4],
                                  semg.at[slot]).wait()
            kp = (k + 2) % 4
            if k < 2:
                lin_issue(i + 2, kp)
            else:
                @pl.when(g < (CHUNKS // 4) - 1)
                def _p():
                    lin_issue(i + 2, kp)

            def edge_body(q, ecarry):
                cw = ecarry
                e0 = q * 4
                eav = eab[slot, pl.ds(e0, 16)]
                for u in range(4):
                    e = e0 + u
                    ea_s = eav[u]
                    for v in range(4):
                        yv = rows[k4, e, pl.ds(v * 16, 16)]
                        m = jnp.maximum(yv + ea_s * cw[v], 0.0)
                        ex = jnp.exp(m)
                        outb[k4, e, pl.ds(v * 16, 16)] = m * ex
                        outb[k4, e, pl.ds(64 + v * 16, 16)] = ex
                return ecarry

            lax.fori_loop(0, CHUNK // 4, edge_body, (w0, w1, w2, w3))
            pltpu.async_copy(outb.at[k4], shared_acc.at[dstb.at[slot]],
                             sems.at[slot], add=True)
            if k < 2:
                lin_wait_and_gather(i + 2, kp)
            else:
                @pl.when(g < (CHUNKS // 4) - 1)
                def _q():
                    lin_wait_and_gather(i + 2, kp)
        return carry

    lax.fori_loop(0, CHUNKS // 4, group_body, 0)
    for k in range(2, 4):
        scatter_wait(k)
    plsc.subcore_barrier()
    rb = s * ROWS_PER_TILE
    pltpu.sync_copy(shared_acc.at[pl.ds(rb, ROWS_PER_TILE)],
                    acc_out.at[c, pl.ds(rb, ROWS_PER_TILE)])

    @pl.when(s == NS - 1)
    def _copy_tail():
        pltpu.sync_copy(shared_acc.at[pl.ds(NS * ROWS_PER_TILE, 16)],
                        acc_out.at[c, pl.ds(NS * ROWS_PER_TILE, 16)])


_edge_pass = functools.partial(
    pl.kernel,
    out_type=jax.ShapeDtypeStruct((NC, N, 128), jnp.float32),
    mesh=plsc.VectorSubcoreMesh(core_axis_name="c", subcore_axis_name="s"),
    compiler_params=pltpu.CompilerParams(use_tc_tiling_on_sc=False),
    scratch_types=[
        pltpu.VMEM_SHARED((N + 16, 128), jnp.float32),
        pltpu.VMEM((4, CHUNK), jnp.int32),
        pltpu.VMEM((4, CHUNK), jnp.int32),
        pltpu.VMEM((4, CHUNK + 16), jnp.float32),
        pltpu.VMEM((2, CHUNK, 64), jnp.float32),
        pltpu.VMEM((2, CHUNK, 128), jnp.float32),
        pltpu.VMEM((64,), jnp.float32),
        pltpu.SemaphoreType.DMA((4,)),
        pltpu.SemaphoreType.DMA((4,)),
        pltpu.SemaphoreType.DMA((4,)),
    ],
)(_edge_body)


def kernel(node_x, in_degree, out_degree, edge_index, edge_attr, batch,
           node_enc, in_deg_enc, out_deg_enc, ln_g, ln_b, wl_W, wl_b,
           mlp_W, mlp_b, r0_W, r0_b, r1_W, r1_b, r2_W, r2_b, r3_W, r3_b):
    x = jnp.take(node_enc, node_x, axis=0).sum(axis=-2) \
        + jnp.take(in_deg_enc, in_degree, axis=0) \
        + jnp.take(out_deg_enc, out_degree, axis=0)

    pad = EP - E
    srcp = jnp.concatenate(
        [edge_index[0].astype(jnp.int32), jnp.zeros((pad,), jnp.int32)])
    dstp = jnp.concatenate(
        [edge_index[1].astype(jnp.int32), jnp.full((pad,), N, jnp.int32)])
    eap = jnp.concatenate([edge_attr, jnp.zeros((pad,), jnp.float32)])

    outs = []
    for l in range(L):
        y = _layer_norm(x, ln_g[l], ln_b[l])
        y2 = jnp.concatenate(
            [y[:, :64] + wl_b[l, :64][None, :],
             y[:, 64:] + wl_b[l, 64:][None, :]], axis=0)
        wb = jnp.stack([wl_W[l, 0, :64], wl_W[l, 0, 64:]])
        acc = _edge_pass(y2, srcp, dstp, eap, wb)
        num = jnp.concatenate([acc[0, :, :64], acc[1, :, :64]], axis=1)
        den = jnp.concatenate([acc[0, :, 64:], acc[1, :, 64:]], axis=1)
        aggr = num / (den + 1e-16)
        out = (aggr + y) @ mlp_W[l] + mlp_b[l]
        x = jax.nn.relu(out)
        outs.append(jax.ops.segment_sum(x, batch, num_segments=G))
    h = jnp.concatenate(outs, axis=1)

    out = pl.pallas_call(
        _readout_kernel,
        out_shape=jax.ShapeDtypeStruct((G, 1), jnp.float32),
    )(h, r0_W, r0_b, r1_W, r1_b, r2_W, r2_b, r3_W, r3_b)
    return out


# speedup vs baseline: 8.3252x; 2.2808x over previous
import functools

import jax
import jax.numpy as jnp
from jax import lax
from jax.experimental import pallas as pl
from jax.experimental.pallas import tpu as pltpu
from jax.experimental.pallas import tpu_sc as plsc

N = 10000
E = 320000
D = 128
L = 6
G = 64

NS = 16
NC = 2
CHUNK = 128
CHUNKS = 160
EP = CHUNKS * CHUNK * NS
ROWS_PER_TILE = 624


def _layer_norm(x, g, b, eps=1e-5):
    mu = jnp.mean(x, axis=-1, keepdims=True)
    var = jnp.mean((x - mu) ** 2, axis=-1, keepdims=True)
    return (x - mu) / jnp.sqrt(var + eps) * g + b


def _gelu(x):
    return 0.5 * x * (1.0 + jax.lax.erf(x * 0.7071067811865476))


def _readout_kernel(h_ref, w0, b0, w1, b1, w2, b2, w3, b3, out_ref):
    h = h_ref[...]
    h = _gelu(h @ w0[...] + b0[...][None, :])
    h = _gelu(h @ w1[...] + b1[...][None, :])
    h = _gelu(h @ w2[...] + b2[...][None, :])
    out_ref[...] = h @ w3[...] + b3[...][None, :]


def _edge_body(y2, srcp, dstp, eap, wb, acc_out,
               shared_acc, idxb, dstb, eab, rows, outb, wbv,
               semg, sems, seml):
    c = lax.axis_index("c")
    s = lax.axis_index("s")
    pltpu.sync_copy(wb.at[c], wbv)
    cN = c * N

    def lin_issue(j, slot):
        base = (s * CHUNKS + j) * CHUNK
        pltpu.async_copy(srcp.at[pl.ds(base, CHUNK)], idxb.at[slot],
                         seml.at[slot])
        pltpu.async_copy(dstp.at[pl.ds(base, CHUNK)], dstb.at[slot],
                         seml.at[slot])
        pltpu.async_copy(eap.at[pl.ds(base, CHUNK)],
                         eab.at[slot, pl.ds(0, CHUNK)], seml.at[slot])

    def lin_wait_and_gather(j, slot):
        base = (s * CHUNKS + j) * CHUNK
        pltpu.make_async_copy(srcp.at[pl.ds(base, CHUNK)], idxb.at[slot],
                              seml.at[slot]).wait()
        pltpu.make_async_copy(dstp.at[pl.ds(base, CHUNK)], dstb.at[slot],
                              seml.at[slot]).wait()
        pltpu.make_async_copy(eap.at[pl.ds(base, CHUNK)],
                              eab.at[slot, pl.ds(0, CHUNK)],
                              seml.at[slot]).wait()
        for t in range(8):
            idxb[slot, pl.ds(t * 16, 16)] = \
                idxb[slot, pl.ds(t * 16, 16)] + cN
        pltpu.async_copy(y2.at[idxb.at[slot]], rows.at[slot % 2],
                         semg.at[slot])

    for k in range(2):
        lin_issue(k, k)

    z16 = jnp.zeros((16,), jnp.float32)

    def zloop(t, carry):
        for v in range(8):
            outb[0, t, pl.ds(v * 16, 16)] = z16
        return carry

    lax.fori_loop(0, 128, zloop, 0)
    zb = s * ROWS_PER_TILE
    for k in range(4):
        pltpu.sync_copy(outb.at[0, pl.ds(0, 128)],
                        shared_acc.at[pl.ds(zb + k * 128, 128)])
    pltpu.sync_copy(outb.at[0, pl.ds(0, ROWS_PER_TILE - 512)],
                    shared_acc.at[pl.ds(zb + 512, ROWS_PER_TILE - 512)])

    @pl.when(s == NS - 1)
    def _zero_tail():
        pltpu.sync_copy(outb.at[0, pl.ds(0, 32)],
                        shared_acc.at[pl.ds(NS * ROWS_PER_TILE, 32)])

    for k in range(2):
        lin_wait_and_gather(k, k)

    plsc.subcore_barrier()

    def scatter_wait(slot):
        pltpu.make_async_copy(outb.at[slot % 2],
                              shared_acc.at[dstb.at[slot]],
                              sems.at[slot]).wait()

    w0 = wbv[pl.ds(0, 16)]
    w1 = wbv[pl.ds(16, 16)]
    w2 = wbv[pl.ds(32, 16)]
    w3 = wbv[pl.ds(48, 16)]

    def group_body(g, carry):
        for k in range(4):
            i = 4 * g + k
            slot = k
            k4 = k % 2
            if k >= 2:
                scatter_wait(k - 2)
            else:
                @pl.when(g > 0)
                def _w():
                    scatter_wait(k + 2)
            pltpu.make_async_copy(y2.at[idxb.at[slot]], rows.at[k4],
                                  semg.at[slot]).wait()
            kp = (k + 2) % 4
            if k < 2:
                lin_issue(i + 2, kp)
            else:
                @pl.when(g < (CHUNKS // 4) - 1)
                def _p():
                    lin_issue(i + 2, kp)

            @plsc.parallel_loop(0, CHUNK // 4, unroll=1,
                                carry=(w0, w1, w2, w3))
            def edge_body(q, ecarry):
                cw = ecarry
                e0 = q * 4
                eav = eab[slot, pl.ds(e0, 16)]
                for u in range(4):
                    e = e0 + u
                    ea_s = eav[u]
                    for v in range(4):
                        yv = rows[k4, e, pl.ds(v * 16, 16)]
                        m = jnp.maximum(yv + ea_s * cw[v], 0.0)
                        ex = jnp.exp(m)
                        outb[k4, e, pl.ds(v * 16, 16)] = m * ex
                        outb[k4, e, pl.ds(64 + v * 16, 16)] = ex
                return ecarry
            pltpu.async_copy(outb.at[k4], shared_acc.at[dstb.at[slot]],
                             sems.at[slot], add=True)
            if k < 2:
                lin_wait_and_gather(i + 2, kp)
            else:
                @pl.when(g < (CHUNKS // 4) - 1)
                def _q():
                    lin_wait_and_gather(i + 2, kp)
        return carry

    lax.fori_loop(0, CHUNKS // 4, group_body, 0)
    for k in range(2, 4):
        scatter_wait(k)
    plsc.subcore_barrier()
    rb = s * ROWS_PER_TILE
    pltpu.sync_copy(shared_acc.at[pl.ds(rb, ROWS_PER_TILE)],
                    acc_out.at[c, pl.ds(rb, ROWS_PER_TILE)])

    @pl.when(s == NS - 1)
    def _copy_tail():
        pltpu.sync_copy(shared_acc.at[pl.ds(NS * ROWS_PER_TILE, 16)],
                        acc_out.at[c, pl.ds(NS * ROWS_PER_TILE, 16)])


_edge_pass = functools.partial(
    pl.kernel,
    out_type=jax.ShapeDtypeStruct((NC, N, 128), jnp.float32),
    mesh=plsc.VectorSubcoreMesh(core_axis_name="c", subcore_axis_name="s"),
    compiler_params=pltpu.CompilerParams(use_tc_tiling_on_sc=False),
    scratch_types=[
        pltpu.VMEM_SHARED((N + 16, 128), jnp.float32),
        pltpu.VMEM((4, CHUNK), jnp.int32),
        pltpu.VMEM((4, CHUNK), jnp.int32),
        pltpu.VMEM((4, CHUNK + 16), jnp.float32),
        pltpu.VMEM((2, CHUNK, 64), jnp.float32),
        pltpu.VMEM((2, CHUNK, 128), jnp.float32),
        pltpu.VMEM((64,), jnp.float32),
        pltpu.SemaphoreType.DMA((4,)),
        pltpu.SemaphoreType.DMA((4,)),
        pltpu.SemaphoreType.DMA((4,)),
    ],
)(_edge_body)


def kernel(node_x, in_degree, out_degree, edge_index, edge_attr, batch,
           node_enc, in_deg_enc, out_deg_enc, ln_g, ln_b, wl_W, wl_b,
           mlp_W, mlp_b, r0_W, r0_b, r1_W, r1_b, r2_W, r2_b, r3_W, r3_b):
    x = jnp.take(node_enc, node_x, axis=0).sum(axis=-2) \
        + jnp.take(in_deg_enc, in_degree, axis=0) \
        + jnp.take(out_deg_enc, out_degree, axis=0)

    pad = EP - E
    srcp = jnp.concatenate(
        [edge_index[0].astype(jnp.int32), jnp.zeros((pad,), jnp.int32)])
    dstp = jnp.concatenate(
        [edge_index[1].astype(jnp.int32), jnp.full((pad,), N, jnp.int32)])
    eap = jnp.concatenate([edge_attr, jnp.zeros((pad,), jnp.float32)])

    outs = []
    for l in range(L):
        y = _layer_norm(x, ln_g[l], ln_b[l])
        y2 = jnp.concatenate(
            [y[:, :64] + wl_b[l, :64][None, :],
             y[:, 64:] + wl_b[l, 64:][None, :]], axis=0)
        wb = jnp.stack([wl_W[l, 0, :64], wl_W[l, 0, 64:]])
        acc = _edge_pass(y2, srcp, dstp, eap, wb)
        num = jnp.concatenate([acc[0, :, :64], acc[1, :, :64]], axis=1)
        den = jnp.concatenate([acc[0, :, 64:], acc[1, :, 64:]], axis=1)
        aggr = num / (den + 1e-16)
        out = (aggr + y) @ mlp_W[l] + mlp_b[l]
        x = jax.nn.relu(out)
        outs.append(jax.ops.segment_sum(x, batch, num_segments=G))
    h = jnp.concatenate(outs, axis=1)

    out = pl.pallas_call(
        _readout_kernel,
        out_shape=jax.ShapeDtypeStruct((G, 1), jnp.float32),
    )(h, r0_W, r0_b, r1_W, r1_b, r2_W, r2_b, r3_W, r3_b)
    return out


# speedup vs baseline: 11.8454x; 1.4228x over previous
import functools

import jax
import jax.numpy as jnp
from jax import lax
from jax.experimental import pallas as pl
from jax.experimental.pallas import tpu as pltpu
from jax.experimental.pallas import tpu_sc as plsc

N = 10000
E = 320000
D = 128
L = 6
G = 64
NUM_NODE_TYPE = 25
NUM_DEG = 256

NS = 16
NC = 2
CHUNK = 128
CHUNKS = 160
EP = CHUNKS * CHUNK * NS
ROWS_PER_TILE = 624

RB = 400
NB = N // RB


def _edge_body(y2, srcp2, dstp, eap, wb, acc_out,
               shared_acc, idxb, dstb, eab, rows, outb, wbv,
               semg, sems, seml):
    c = lax.axis_index("c")
    s = lax.axis_index("s")
    pltpu.sync_copy(wb.at[c], wbv)

    def lin_issue(j, slot):
        base = (s * CHUNKS + j) * CHUNK
        pltpu.async_copy(srcp2.at[c, pl.ds(base, CHUNK)], idxb.at[slot],
                         seml.at[slot])
        pltpu.async_copy(dstp.at[pl.ds(base, CHUNK)], dstb.at[slot],
                         seml.at[slot])
        pltpu.async_copy(eap.at[pl.ds(base, CHUNK)],
                         eab.at[slot, pl.ds(0, CHUNK)], seml.at[slot])

    def lin_wait_and_gather(j, slot):
        base = (s * CHUNKS + j) * CHUNK
        pltpu.make_async_copy(srcp2.at[c, pl.ds(base, CHUNK)], idxb.at[slot],
                              seml.at[slot]).wait()
        pltpu.make_async_copy(dstp.at[pl.ds(base, CHUNK)], dstb.at[slot],
                              seml.at[slot]).wait()
        pltpu.make_async_copy(eap.at[pl.ds(base, CHUNK)],
                              eab.at[slot, pl.ds(0, CHUNK)],
                              seml.at[slot]).wait()
        pltpu.async_copy(y2.at[idxb.at[slot]], rows.at[slot % 2],
                         semg.at[slot])

    for k in range(2):
        lin_issue(k, k)

    z16 = jnp.zeros((16,), jnp.float32)

    def zloop(t, carry):
        for v in range(8):
            outb[0, t, pl.ds(v * 16, 16)] = z16
        return carry

    lax.fori_loop(0, 128, zloop, 0)
    zb = s * ROWS_PER_TILE
    for k in range(4):
        pltpu.sync_copy(outb.at[0, pl.ds(0, 128)],
                        shared_acc.at[pl.ds(zb + k * 128, 128)])
    pltpu.sync_copy(outb.at[0, pl.ds(0, ROWS_PER_TILE - 512)],
                    shared_acc.at[pl.ds(zb + 512, ROWS_PER_TILE - 512)])

    @pl.when(s == NS - 1)
    def _zero_tail():
        pltpu.sync_copy(outb.at[0, pl.ds(0, 32)],
                        shared_acc.at[pl.ds(NS * ROWS_PER_TILE, 32)])

    for k in range(2):
        lin_wait_and_gather(k, k)

    plsc.subcore_barrier()

    def scatter_wait(slot):
        pltpu.make_async_copy(outb.at[slot % 2],
                              shared_acc.at[dstb.at[slot]],
                              sems.at[slot]).wait()

    w0 = wbv[pl.ds(0, 16)]
    w1 = wbv[pl.ds(16, 16)]
    w2 = wbv[pl.ds(32, 16)]
    w3 = wbv[pl.ds(48, 16)]

    def group_body(g, carry):
        for k in range(4):
            i = 4 * g + k
            slot = k
            k4 = k % 2
            if k >= 2:
                scatter_wait(k - 2)
            else:
                @pl.when(g > 0)
                def _w():
                    scatter_wait(k + 2)
            pltpu.make_async_copy(y2.at[idxb.at[slot]], rows.at[k4],
                                  semg.at[slot]).wait()
            kp = (k + 2) % 4
            if k < 2:
                lin_issue(i + 2, kp)
            else:
                @pl.when(g < (CHUNKS // 4) - 1)
                def _p():
                    lin_issue(i + 2, kp)

            @plsc.parallel_loop(0, CHUNK // 4, unroll=1,
                                carry=(w0, w1, w2, w3))
            def edge_body(q, ecarry):
                cw = ecarry
                e0 = q * 4
                eav = eab[slot, pl.ds(e0, 16)]
                for u in range(4):
                    e = e0 + u
                    ea_s = eav[u]
                    for v in range(4):
                        yv = rows[k4, e, pl.ds(v * 16, 16)]
                        m = jnp.maximum(yv + ea_s * cw[v], 0.0)
                        ex = jnp.exp(m)
                        outb[k4, e, pl.ds(v * 16, 16)] = m * ex
                        outb[k4, e, pl.ds(64 + v * 16, 16)] = ex
                return ecarry

            pltpu.async_copy(outb.at[k4], shared_acc.at[dstb.at[slot]],
                             sems.at[slot], add=True)
            if k < 2:
                lin_wait_and_gather(i + 2, kp)
            else:
                @pl.when(g < (CHUNKS // 4) - 1)
                def _q():
                    lin_wait_and_gather(i + 2, kp)
        return carry

    lax.fori_loop(0, CHUNKS // 4, group_body, 0)
    for k in range(2, 4):
        scatter_wait(k)
    plsc.subcore_barrier()
    rb = s * ROWS_PER_TILE
    pltpu.sync_copy(shared_acc.at[pl.ds(rb, ROWS_PER_TILE)],
                    acc_out.at[c, pl.ds(rb, ROWS_PER_TILE)])

    @pl.when(s == NS - 1)
    def _copy_tail():
        pltpu.sync_copy(shared_acc.at[pl.ds(NS * ROWS_PER_TILE, 16)],
                        acc_out.at[c, pl.ds(NS * ROWS_PER_TILE, 16)])


_edge_pass = functools.partial(
    pl.kernel,
    out_type=jax.ShapeDtypeStruct((NC, N, 128), jnp.float32),
    mesh=plsc.VectorSubcoreMesh(core_axis_name="c", subcore_axis_name="s"),
    compiler_params=pltpu.CompilerParams(use_tc_tiling_on_sc=False),
    scratch_types=[
        pltpu.VMEM_SHARED((N + 16, 128), jnp.float32),
        pltpu.VMEM((4, CHUNK), jnp.int32),
        pltpu.VMEM((4, CHUNK), jnp.int32),
        pltpu.VMEM((4, CHUNK + 16), jnp.float32),
        pltpu.VMEM((2, CHUNK, 64), jnp.float32),
        pltpu.VMEM((2, CHUNK, 128), jnp.float32),
        pltpu.VMEM((64,), jnp.float32),
        pltpu.SemaphoreType.DMA((4,)),
        pltpu.SemaphoreType.DMA((4,)),
        pltpu.SemaphoreType.DMA((4,)),
    ],
)(_edge_body)


def _ln(x, g, b):
    mu = jnp.mean(x, axis=-1, keepdims=True)
    var = jnp.mean((x - mu) ** 2, axis=-1, keepdims=True)
    return (x - mu) / jnp.sqrt(var + 1e-5) * g + b


def _encoder_kernel(nxt_ref, ind_ref, outd_ref, nenc_ref, ienc_ref, oenc_ref,
                    g0_ref, b0_ref, wlb0_ref, y_ref, y2_ref):
    nx = nxt_ref[0]
    iota_t = lax.broadcasted_iota(jnp.int32, (NUM_NODE_TYPE, RB), 0)
    counts = jnp.zeros((NUM_NODE_TYPE, RB), jnp.float32)
    for k in range(9):
        counts += jnp.where(nx[k][None, :] == iota_t, 1.0, 0.0)
    x = lax.dot_general(counts, nenc_ref[...], (((0,), (0,)), ((), ())),
                        preferred_element_type=jnp.float32)
    iota_d = lax.broadcasted_iota(jnp.int32, (NUM_DEG, RB), 0)
    ohi = jnp.where(ind_ref[0, 0][None, :] == iota_d, 1.0, 0.0)
    oho = jnp.where(outd_ref[0, 0][None, :] == iota_d, 1.0, 0.0)
    x = x + lax.dot_general(ohi, ienc_ref[...], (((0,), (0,)), ((), ())),
                            preferred_element_type=jnp.float32)
    x = x + lax.dot_general(oho, oenc_ref[...], (((0,), (0,)), ((), ())),
                            preferred_element_type=jnp.float32)
    y = _ln(x, g0_ref[...], b0_ref[...])
    y_ref[...] = y
    wlb = wlb0_ref[...]
    y2_ref[0, :, :] = y[:, :64] + wlb[0, :64][None, :]
    y2_ref[1, :, :] = y[:, 64:] + wlb[0, 64:][None, :]


def _layer_kernel(acc_ref, y_ref, batch_ref, mw_ref, mb_ref,
                  gn_ref, bn_ref, wlbn_ref, pool_ref, yn_ref, y2n_ref):
    num = jnp.concatenate([acc_ref[0, :, :64], acc_ref[1, :, :64]], axis=1)
    den = jnp.concatenate([acc_ref[0, :, 64:], acc_ref[1, :, 64:]], axis=1)
    z = num / (den + 1e-16) + y_ref[...]
    out = jnp.maximum(z @ mw_ref[...] + mb_ref[...], 0.0)
    oh = jnp.where(batch_ref[0, 0][:, None]
                   == lax.broadcasted_iota(jnp.int32, (RB, G), 1), 1.0, 0.0)
    contrib = lax.dot_general(oh, out, (((0,), (0,)), ((), ())),
                              preferred_element_type=jnp.float32)

    @pl.when(pl.program_id(0) == 0)
    def _init():
        pool_ref[...] = contrib

    @pl.when(pl.program_id(0) > 0)
    def _accum():
        pool_ref[...] += contrib

    yn = _ln(out, gn_ref[...], bn_ref[...])
    yn_ref[...] = yn
    wlb = wlbn_ref[...]
    y2n_ref[0, :, :] = yn[:, :64] + wlb[0, :64][None, :]
    y2n_ref[1, :, :] = yn[:, 64:] + wlb[0, 64:][None, :]


def _gelu(x):
    return 0.5 * x * (1.0 + lax.erf(x * 0.7071067811865476))


def _readout_kernel(h_ref, w0, b0, w1, b1, w2, b2, w3, b3, out_ref):
    h = h_ref[...]
    h = _gelu(h @ w0[...] + b0[...][None, :])
    h = _gelu(h @ w1[...] + b1[...][None, :])
    h = _gelu(h @ w2[...] + b2[...][None, :])
    out_ref[...] = h @ w3[...] + b3[...][None, :]


_encoder_call = pl.pallas_call(
    _encoder_kernel,
    grid=(NB,),
    in_specs=[
        pl.BlockSpec((1, 9, RB), lambda i: (i, 0, 0)),
        pl.BlockSpec((1, 1, RB), lambda i: (i, 0, 0)),
        pl.BlockSpec((1, 1, RB), lambda i: (i, 0, 0)),
        pl.BlockSpec((NUM_NODE_TYPE, D), lambda i: (0, 0)),
        pl.BlockSpec((NUM_DEG, D), lambda i: (0, 0)),
        pl.BlockSpec((NUM_DEG, D), lambda i: (0, 0)),
        pl.BlockSpec((1, D), lambda i: (0, 0)),
        pl.BlockSpec((1, D), lambda i: (0, 0)),
        pl.BlockSpec((1, D), lambda i: (0, 0)),
    ],
    out_specs=[
        pl.BlockSpec((RB, D), lambda i: (i, 0)),
        pl.BlockSpec((2, RB, 64), lambda i: (0, i, 0)),
    ],
    out_shape=[
        jax.ShapeDtypeStruct((N, D), jnp.float32),
        jax.ShapeDtypeStruct((2, N, 64), jnp.float32),
    ],
)

_layer_call = pl.pallas_call(
    _layer_kernel,
    grid=(NB,),
    in_specs=[
        pl.BlockSpec((2, RB, 128), lambda i: (0, i, 0)),
        pl.BlockSpec((RB, D), lambda i: (i, 0)),
        pl.BlockSpec((1, 1, RB), lambda i: (i, 0, 0)),
        pl.BlockSpec((D, D), lambda i: (0, 0)),
        pl.BlockSpec((1, D), lambda i: (0, 0)),
        pl.BlockSpec((1, D), lambda i: (0, 0)),
        pl.BlockSpec((1, D), lambda i: (0, 0)),
        pl.BlockSpec((1, D), lambda i: (0, 0)),
    ],
    out_specs=[
        pl.BlockSpec((G, D), lambda i: (0, 0)),
        pl.BlockSpec((RB, D), lambda i: (i, 0)),
        pl.BlockSpec((2, RB, 64), lambda i: (0, i, 0)),
    ],
    out_shape=[
        jax.ShapeDtypeStruct((G, D), jnp.float32),
        jax.ShapeDtypeStruct((N, D), jnp.float32),
        jax.ShapeDtypeStruct((2, N, 64), jnp.float32),
    ],
)


def kernel(node_x, in_degree, out_degree, edge_index, edge_attr, batch,
           node_enc, in_deg_enc, out_deg_enc, ln_g, ln_b, wl_W, wl_b,
           mlp_W, mlp_b, r0_W, r0_b, r1_W, r1_b, r2_W, r2_b, r3_W, r3_b):
    nxt = node_x.astype(jnp.int32).reshape(NB, RB, 9).transpose(0, 2, 1)
    ind3 = in_degree.astype(jnp.int32).reshape(NB, 1, RB)
    outd3 = out_degree.astype(jnp.int32).reshape(NB, 1, RB)
    batch3 = batch.astype(jnp.int32).reshape(NB, 1, RB)

    pad = EP - E
    srcp = jnp.concatenate(
        [edge_index[0].astype(jnp.int32), jnp.zeros((pad,), jnp.int32)])
    srcp2 = jnp.stack([srcp, srcp + N])
    dstp = jnp.concatenate(
        [edge_index[1].astype(jnp.int32), jnp.full((pad,), N, jnp.int32)])
    eap = jnp.concatenate([edge_attr, jnp.zeros((pad,), jnp.float32)])

    y, y2 = _encoder_call(nxt, ind3, outd3, node_enc, in_deg_enc, out_deg_enc,
                          ln_g[0][None], ln_b[0][None], wl_b[0][None])

    pools = []
    for l in range(L):
        wb = jnp.stack([wl_W[l, 0, :64], wl_W[l, 0, 64:]])
        acc = _edge_pass(y2.reshape(2 * N, 64), srcp2, dstp, eap, wb)
        nl = min(l + 1, L - 1)
        pool, y, y2 = _layer_call(acc, y, batch3, mlp_W[l], mlp_b[l][None],
                                  ln_g[nl][None], ln_b[nl][None],
                                  wl_b[nl][None])
        pools.append(pool)
    h = jnp.concatenate(pools, axis=1)

    out = pl.pallas_call(
        _readout_kernel,
        out_shape=jax.ShapeDtypeStruct((G, 1), jnp.float32),
    )(h, r0_W, r0_b, r1_W, r1_b, r2_W, r2_b, r3_W, r3_b)
    return out
